# baseline (speedup 1.0000x reference)
import functools

import numpy as np

import jax
import jax.numpy as jnp
from jax.experimental import pallas as pl
from jax.experimental.pallas import tpu as pltpu


def _params(sem):
    return pltpu.CompilerParams(dimension_semantics=sem,
                                vmem_limit_bytes=60 * 1024 * 1024)


def _sel_matrix(hw):
    h2 = hw // 2
    s = np.zeros((h2 * h2, hw * hw), np.float32)
    for i in range(h2):
        for j in range(h2):
            s[i * h2 + j, (2 * i) * hw + 2 * j] = 1.0
    return jnp.asarray(s, jnp.bfloat16)


def _im2col_dot(rows_ref, im_ref, w_ref, b_ref, th, wp, c):
    for dh in range(3):
        for dw in range(3):
            t = dh * 3 + dw
            im_ref[0:th * wp, t * c:(t + 1) * c] = (
                rows_ref[dh:dh + th, dw:dw + wp, :].reshape(th * wp, c))
    return (jnp.dot(im_ref[0:th * wp, :], w_ref[0],
                    preferred_element_type=jnp.float32) + b_ref[0])


def _to_nchw(o_ref, res, th, wp):
    for r in range(th):
        o_ref[0, :, r, :] = jnp.transpose(
            res[r * wp:(r + 1) * wp, :], (1, 0)).astype(o_ref.dtype)


def _updup(s, take):
    k, w, c = s.shape
    d = jnp.concatenate([s[:, None], s[:, None]], axis=1)
    hd = d.reshape(2 * k, w, c)[1:1 + take]
    flat = hd.reshape(take * w, c)
    wd = jnp.concatenate([flat, flat], axis=-1)
    return wd.reshape(take * w * 2, c)


def _p5_kernel(x_ref, w1_ref, b1_ref, w3a_ref, w3b_ref, w3c_ref,
               b3a_ref, b3b_ref, b3c_ref, s6_ref, s7_ref,
               lat5_ref, o5_ref, o6_ref, o7_ref,
               rows_ref, im_ref, rows7_ref):
    x = x_ref[0].astype(jnp.bfloat16)
    w1 = w1_ref[...].astype(jnp.bfloat16)
    lat_t = jnp.dot(w1, x, preferred_element_type=jnp.float32) + b1_ref[...]
    lat5 = jnp.transpose(lat_t, (1, 0)).astype(jnp.bfloat16)
    lat5_ref[0] = lat5

    rows_ref[...] = jnp.zeros_like(rows_ref)
    rows_ref[1:33, 1:33, :] = lat5.reshape(32, 32, 256)
    out5 = _im2col_dot(rows_ref, im_ref, w3a_ref, b3a_ref, 32, 32, 256)
    _to_nchw(o5_ref, out5, 32, 32)

    rows_ref[1:33, 1:33, :] = out5.reshape(32, 32, 256).astype(jnp.bfloat16)
    p6full = _im2col_dot(rows_ref, im_ref, w3b_ref, b3b_ref, 32, 32, 256)
    p6 = jnp.dot(s6_ref[...], p6full.astype(jnp.bfloat16),
                 preferred_element_type=jnp.float32)
    _to_nchw(o6_ref, p6, 16, 16)

    rows7_ref[...] = jnp.zeros_like(rows7_ref)
    rows7_ref[1:17, 1:17, :] = p6.reshape(16, 16, 256).astype(jnp.bfloat16)
    p7full = _im2col_dot(rows7_ref, im_ref, w3c_ref, b3c_ref, 16, 16, 256)
    p7 = jnp.dot(s7_ref[...], p7full.astype(jnp.bfloat16),
                 preferred_element_type=jnp.float32)
    _to_nchw(o7_ref, p7, 8, 8)


def _call_a(f5, w1, b1, w3c, b3s, s6, s7):
    return pl.pallas_call(
        _p5_kernel,
        out_shape=[
            jax.ShapeDtypeStruct((2, 1024, 256), jnp.bfloat16),
            jax.ShapeDtypeStruct((2, 256, 32, 32), jnp.float32),
            jax.ShapeDtypeStruct((2, 256, 16, 16), jnp.float32),
            jax.ShapeDtypeStruct((2, 256, 8, 8), jnp.float32),
        ],
        grid=(2,),
        in_specs=[
            pl.BlockSpec((1, 2048, 1024), lambda n: (n, 0, 0)),
            pl.BlockSpec((256, 2048), lambda n: (0, 0)),
            pl.BlockSpec((256, 1), lambda n: (0, 0)),
            pl.BlockSpec((1, 2304, 256), lambda n: (2, 0, 0)),
            pl.BlockSpec((1, 2304, 256), lambda n: (3, 0, 0)),
            pl.BlockSpec((1, 2304, 256), lambda n: (4, 0, 0)),
            pl.BlockSpec((1, 1, 256), lambda n: (2, 0, 0)),
            pl.BlockSpec((1, 1, 256), lambda n: (3, 0, 0)),
            pl.BlockSpec((1, 1, 256), lambda n: (4, 0, 0)),
            pl.BlockSpec((256, 1024), lambda n: (0, 0)),
            pl.BlockSpec((64, 256), lambda n: (0, 0)),
        ],
        out_specs=[
            pl.BlockSpec((1, 1024, 256), lambda n: (n, 0, 0)),
            pl.BlockSpec((1, 256, 32, 32), lambda n: (n, 0, 0, 0)),
            pl.BlockSpec((1, 256, 16, 16), lambda n: (n, 0, 0, 0)),
            pl.BlockSpec((1, 256, 8, 8), lambda n: (n, 0, 0, 0)),
        ],
        scratch_shapes=[
            pltpu.VMEM((34, 34, 256), jnp.bfloat16),
            pltpu.VMEM((1024, 2304), jnp.bfloat16),
            pltpu.VMEM((18, 18, 256), jnp.bfloat16),
        ],
        compiler_params=_params(("parallel",)),
        cost_estimate=pl.CostEstimate(
            flops=2 * 2 * (1024 * 2048 * 256 + 3 * 1024 * 2304 * 256),
            transcendentals=0,
            bytes_accessed=2 * (2048 * 1024 * 4 + 4 * 1024 * 256 * 4)),
    )(f5, w1, b1, w3c, w3c, w3c, b3s, b3s, b3s, s6, s7)


def _p4_kernel(xm_ref, hl_ref, hr_ref, lat5_ref, w1_ref, b1_ref,
               w3_ref, b3_ref, sum4_ref, o4_ref, rows_ref, im_ref):
    l = pl.program_id(1)
    nl = pl.num_programs(1)
    w1 = w1_ref[...].astype(jnp.bfloat16)
    b1 = b1_ref[...]

    xm = xm_ref[0].astype(jnp.bfloat16)
    xl = hl_ref[0, :, 64:128].astype(jnp.bfloat16)
    xr = hr_ref[0, :, 0:64].astype(jnp.bfloat16)

    lm = jnp.dot(w1, xm, preferred_element_type=jnp.float32) + b1
    ll = jnp.dot(w1, xl, preferred_element_type=jnp.float32) + b1
    lr = jnp.dot(w1, xr, preferred_element_type=jnp.float32) + b1
    ll = jnp.where(l == 0, 0.0, ll)
    lr = jnp.where(l == nl - 1, 0.0, lr)
    lat4 = jnp.concatenate([
        jnp.transpose(ll, (1, 0)),
        jnp.transpose(lm, (1, 0)),
        jnp.transpose(lr, (1, 0)),
    ], axis=0)

    lat5 = lat5_ref[0].reshape(32, 32, 256)
    z = jnp.zeros((1, 32, 256), jnp.bfloat16)
    s18 = jnp.where(l == 0,
                    jnp.concatenate([z, lat5[0:17]], axis=0),
                    jnp.concatenate([lat5[15:32], z], axis=0))
    up = _updup(s18, 34)

    sum4 = lat4 + up.astype(jnp.float32)
    ri = jax.lax.broadcasted_iota(jnp.int32, (2176, 1), 0) // 64
    edge = ((ri == 0) & (l == 0)) | ((ri == 33) & (l == nl - 1))
    sum4 = jnp.where(edge, 0.0, sum4).astype(jnp.bfloat16)
    sum4_ref[0] = sum4.reshape(34, 64, 256)[1:33]

    rows_ref[...] = jnp.zeros_like(rows_ref)
    rows_ref[:, 1:65, :] = sum4.reshape(34, 64, 256)
    out4 = _im2col_dot(rows_ref, im_ref, w3_ref, b3_ref, 32, 64, 256)
    _to_nchw(o4_ref, out4, 32, 64)


def _call_b(f4, lat5, w1, b1, w3c, b3s):
    return pl.pallas_call(
        _p4_kernel,
        out_shape=[
            jax.ShapeDtypeStruct((2, 64, 64, 256), jnp.bfloat16),
            jax.ShapeDtypeStruct((2, 256, 64, 64), jnp.float32),
        ],
        grid=(2, 2),
        in_specs=[
            pl.BlockSpec((1, 1024, 2048), lambda n, l: (n, 0, l)),
            pl.BlockSpec((1, 1024, 128),
                         lambda n, l: (n, 0, jnp.maximum(l * 16 - 1, 0))),
            pl.BlockSpec((1, 1024, 128),
                         lambda n, l: (n, 0, jnp.minimum((l + 1) * 16, 31))),
            pl.BlockSpec((1, 1024, 256), lambda n, l: (n, 0, 0)),
            pl.BlockSpec((256, 1024), lambda n, l: (0, 0)),
            pl.BlockSpec((256, 1), lambda n, l: (0, 0)),
            pl.BlockSpec((1, 2304, 256), lambda n, l: (1, 0, 0)),
            pl.BlockSpec((1, 1, 256), lambda n, l: (1, 0, 0)),
        ],
        out_specs=[
            pl.BlockSpec((1, 32, 64, 256), lambda n, l: (n, l, 0, 0)),
            pl.BlockSpec((1, 256, 32, 64), lambda n, l: (n, 0, l, 0)),
        ],
        scratch_shapes=[
            pltpu.VMEM((34, 66, 256), jnp.bfloat16),
            pltpu.VMEM((2048, 2304), jnp.bfloat16),
        ],
        compiler_params=_params(("parallel", "parallel")),
        cost_estimate=pl.CostEstimate(
            flops=2 * 2 * (4096 * 1024 * 256 + 4096 * 2304 * 256),
            transcendentals=0,
            bytes_accessed=2 * (1024 * 4096 * 4 + 2 * 4096 * 256 * 4)),
    )(f4, f4, f4, lat5, w1, b1, w3c, b3s)


def _p3_kernel(xm_ref, hl_ref, hr_ref, s4m_ref, s4t_ref, s4b_ref,
               w1_ref, b1_ref, w3_ref, b3_ref, o3_ref, rows_ref, im_ref):
    l = pl.program_id(1)
    nl = pl.num_programs(1)
    w1 = w1_ref[...].astype(jnp.bfloat16)
    b1 = b1_ref[...]

    xm = xm_ref[0].astype(jnp.bfloat16)
    xl = hl_ref[0].astype(jnp.bfloat16)
    xr = hr_ref[0].astype(jnp.bfloat16)
    lm = jnp.dot(w1, xm, preferred_element_type=jnp.float32) + b1
    ll = jnp.dot(w1, xl, preferred_element_type=jnp.float32) + b1
    lr = jnp.dot(w1, xr, preferred_element_type=jnp.float32) + b1
    ll = jnp.where(l == 0, 0.0, ll)
    lr = jnp.where(l == nl - 1, 0.0, lr)
    lat3 = jnp.concatenate([
        jnp.transpose(ll, (1, 0)),
        jnp.transpose(lm, (1, 0)),
        jnp.transpose(lr, (1, 0)),
    ], axis=0)

    zt = jnp.where(l == 0, 0.0, s4t_ref[0]).astype(jnp.bfloat16)
    zb = jnp.where(l == nl - 1, 0.0, s4b_ref[0]).astype(jnp.bfloat16)
    s10 = jnp.concatenate([zt, s4m_ref[0], zb], axis=0)
    up = _updup(s10, 18)

    sum3 = lat3 + up.astype(jnp.float32)
    ri = jax.lax.broadcasted_iota(jnp.int32, (2304, 1), 0) // 128
    edge = ((ri == 0) & (l == 0)) | ((ri == 17) & (l == nl - 1))
    sum3 = jnp.where(edge, 0.0, sum3).astype(jnp.bfloat16)

    rows_ref[...] = jnp.zeros_like(rows_ref)
    rows_ref[:, 1:129, :] = sum3.reshape(18, 128, 256)
    out3 = _im2col_dot(rows_ref, im_ref, w3_ref, b3_ref, 16, 128, 256)
    _to_nchw(o3_ref, out3, 16, 128)


def _call_c(f3, sum4, w1, b1, w3c, b3s):
    return pl.pallas_call(
        _p3_kernel,
        out_shape=jax.ShapeDtypeStruct((2, 256, 128, 128), jnp.float32),
        grid=(2, 8),
        in_specs=[
            pl.BlockSpec((1, 512, 2048), lambda n, l: (n, 0, l)),
            pl.BlockSpec((1, 512, 128),
                         lambda n, l: (n, 0, jnp.maximum(l * 16 - 1, 0))),
            pl.BlockSpec((1, 512, 128),
                         lambda n, l: (n, 0, jnp.minimum((l + 1) * 16, 127))),
            pl.BlockSpec((1, 8, 64, 256), lambda n, l: (n, l, 0, 0)),
            pl.BlockSpec((1, 1, 64, 256),
                         lambda n, l: (n, jnp.maximum(l * 8 - 1, 0), 0, 0)),
            pl.BlockSpec((1, 1, 64, 256),
                         lambda n, l: (n, jnp.minimum(l * 8 + 8, 63), 0, 0)),
            pl.BlockSpec((256, 512), lambda n, l: (0, 0)),
            pl.BlockSpec((256, 1), lambda n, l: (0, 0)),
            pl.BlockSpec((1, 2304, 256), lambda n, l: (0, 0, 0)),
            pl.BlockSpec((1, 1, 256), lambda n, l: (0, 0, 0)),
        ],
        out_specs=pl.BlockSpec((1, 256, 16, 128), lambda n, l: (n, 0, l, 0)),
        scratch_shapes=[
            pltpu.VMEM((18, 130, 256), jnp.bfloat16),
            pltpu.VMEM((2048, 2304), jnp.bfloat16),
        ],
        compiler_params=_params(("parallel", "parallel")),
        cost_estimate=pl.CostEstimate(
            flops=2 * 2 * (16384 * 512 * 256 + 16384 * 2304 * 256),
            transcendentals=0,
            bytes_accessed=2 * (512 * 16384 * 4 + 16384 * 256 * 4)),
    )(f3, f3, f3, sum4, sum4, sum4, w1, b1, w3c, b3s)


def kernel(feat0, feat1, feat2,
           w1_0, w1_1, w1_2,
           b1_0, b1_1, b1_2,
           w3_0, w3_1, w3_2, w3_3, w3_4,
           b3_0, b3_1, b3_2, b3_3, b3_4):
    C = 256
    w3c = jnp.concatenate(
        [w.reshape(1, C, C, 9) for w in (w3_0, w3_1, w3_2, w3_3, w3_4)],
        axis=0).transpose(0, 3, 2, 1).reshape(5, 9 * C, C).astype(jnp.bfloat16)
    b3s = jnp.concatenate(
        [b.reshape(1, 1, C) for b in (b3_0, b3_1, b3_2, b3_3, b3_4)],
        axis=0).astype(jnp.float32)

    w1m = [w.reshape(C, w.shape[1]) for w in (w1_0, w1_1, w1_2)]
    b1p = [b.reshape(C, 1).astype(jnp.float32) for b in (b1_0, b1_1, b1_2)]

    s6 = _sel_matrix(32)
    s7 = _sel_matrix(16)

    f5 = feat2.reshape(2, 2048, 1024)
    f4 = feat1.reshape(2, 1024, 4096)
    f3 = feat0.reshape(2, 512, 16384)

    lat5, out5, out6, out7 = _call_a(f5, w1m[2], b1p[2], w3c, b3s, s6, s7)
    sum4, out4 = _call_b(f4, lat5, w1m[1], b1p[1], w3c, b3s)
    out3 = _call_c(f3, sum4, w1m[0], b1p[0], w3c, b3s)

    return [out3, out4, out5, out6, out7]


# speedup vs baseline: 1.0034x; 1.0034x over previous
import functools

import numpy as np

import jax
import jax.numpy as jnp
from jax.experimental import pallas as pl
from jax.experimental.pallas import tpu as pltpu


def _params(sem):
    return pltpu.CompilerParams(dimension_semantics=sem,
                                vmem_limit_bytes=60 * 1024 * 1024)


def _sel_matrix(hw):
    h2 = hw // 2
    s = np.zeros((h2 * h2, hw * hw), np.float32)
    for i in range(h2):
        for j in range(h2):
            s[i * h2 + j, (2 * i) * hw + 2 * j] = 1.0
    return jnp.asarray(s, jnp.bfloat16)


def _im2col_dot(rows_ref, im_ref, w_ref, b_ref, th, wp, c):
    for dh in range(3):
        for dw in range(3):
            t = dh * 3 + dw
            im_ref[0:th * wp, t * c:(t + 1) * c] = (
                rows_ref[dh:dh + th, dw:dw + wp, :].reshape(th * wp, c))
    return (jnp.dot(im_ref[0:th * wp, :], w_ref[0],
                    preferred_element_type=jnp.float32) + b_ref[0])


def _to_nchw(o_ref, res, th, wp):
    for r in range(th):
        o_ref[0, :, r, :] = jnp.transpose(
            res[r * wp:(r + 1) * wp, :], (1, 0)).astype(o_ref.dtype)


def _updup(s, take):
    k, w, c = s.shape
    d = jnp.concatenate([s[:, None], s[:, None]], axis=1)
    hd = d.reshape(2 * k, w, c)[1:1 + take]
    flat = hd.reshape(take * w, c)
    wd = jnp.concatenate([flat, flat], axis=-1)
    return wd.reshape(take * w * 2, c)


def _p5_kernel(x_ref, w1_ref, b1_ref, w3a_ref, w3b_ref, w3c_ref,
               b3a_ref, b3b_ref, b3c_ref,
               lat5_ref, o5_ref, o6_ref, o7_ref,
               rows_ref, im_ref, rows7_ref):
    x = x_ref[0].astype(jnp.bfloat16)
    w1 = w1_ref[...].astype(jnp.bfloat16)
    lat_t = jnp.dot(w1, x, preferred_element_type=jnp.float32) + b1_ref[...]
    lat5 = jnp.transpose(lat_t, (1, 0)).astype(jnp.bfloat16)
    lat5_ref[0] = lat5

    rows_ref[...] = jnp.zeros_like(rows_ref)
    rows_ref[1:33, 1:33, :] = lat5.reshape(32, 32, 256)
    out5 = _im2col_dot(rows_ref, im_ref, w3a_ref, b3a_ref, 32, 32, 256)
    _to_nchw(o5_ref, out5, 32, 32)

    rows_ref[1:33, 1:33, :] = out5.reshape(32, 32, 256).astype(jnp.bfloat16)
    p6full = _im2col_dot(rows_ref, im_ref, w3b_ref, b3b_ref, 32, 32, 256)
    p6 = p6full.reshape(16, 2, 32, 256)[:, 0]
    p6 = p6.reshape(16, 16, 2, 256)[:, :, 0, :].reshape(256, 256)
    _to_nchw(o6_ref, p6, 16, 16)

    rows7_ref[...] = jnp.zeros_like(rows7_ref)
    rows7_ref[1:17, 1:17, :] = p6.reshape(16, 16, 256).astype(jnp.bfloat16)
    p7full = _im2col_dot(rows7_ref, im_ref, w3c_ref, b3c_ref, 16, 16, 256)
    p7 = p7full.reshape(8, 2, 16, 256)[:, 0]
    p7 = p7.reshape(8, 8, 2, 256)[:, :, 0, :].reshape(64, 256)
    _to_nchw(o7_ref, p7, 8, 8)


def _call_a(f5, w1, b1, w3c, b3s):
    return pl.pallas_call(
        _p5_kernel,
        out_shape=[
            jax.ShapeDtypeStruct((2, 1024, 256), jnp.bfloat16),
            jax.ShapeDtypeStruct((2, 256, 32, 32), jnp.float32),
            jax.ShapeDtypeStruct((2, 256, 16, 16), jnp.float32),
            jax.ShapeDtypeStruct((2, 256, 8, 8), jnp.float32),
        ],
        grid=(2,),
        in_specs=[
            pl.BlockSpec((1, 2048, 1024), lambda n: (n, 0, 0)),
            pl.BlockSpec((256, 2048), lambda n: (0, 0)),
            pl.BlockSpec((256, 1), lambda n: (0, 0)),
            pl.BlockSpec((1, 2304, 256), lambda n: (2, 0, 0)),
            pl.BlockSpec((1, 2304, 256), lambda n: (3, 0, 0)),
            pl.BlockSpec((1, 2304, 256), lambda n: (4, 0, 0)),
            pl.BlockSpec((1, 1, 256), lambda n: (2, 0, 0)),
            pl.BlockSpec((1, 1, 256), lambda n: (3, 0, 0)),
            pl.BlockSpec((1, 1, 256), lambda n: (4, 0, 0)),
        ],
        out_specs=[
            pl.BlockSpec((1, 1024, 256), lambda n: (n, 0, 0)),
            pl.BlockSpec((1, 256, 32, 32), lambda n: (n, 0, 0, 0)),
            pl.BlockSpec((1, 256, 16, 16), lambda n: (n, 0, 0, 0)),
            pl.BlockSpec((1, 256, 8, 8), lambda n: (n, 0, 0, 0)),
        ],
        scratch_shapes=[
            pltpu.VMEM((34, 34, 256), jnp.bfloat16),
            pltpu.VMEM((1024, 2304), jnp.bfloat16),
            pltpu.VMEM((18, 18, 256), jnp.bfloat16),
        ],
        compiler_params=_params(("parallel",)),
        cost_estimate=pl.CostEstimate(
            flops=2 * 2 * (1024 * 2048 * 256 + 3 * 1024 * 2304 * 256),
            transcendentals=0,
            bytes_accessed=2 * (2048 * 1024 * 4 + 4 * 1024 * 256 * 4)),
    )(f5, w1, b1, w3c, w3c, w3c, b3s, b3s, b3s)


def _p4_kernel(xm_ref, hl_ref, hr_ref, lat5_ref, w1_ref, b1_ref,
               w3_ref, b3_ref, sum4_ref, o4_ref, rows_ref, im_ref):
    l = pl.program_id(1)
    nl = pl.num_programs(1)
    w1 = w1_ref[...].astype(jnp.bfloat16)
    b1 = b1_ref[...]

    xm = xm_ref[0].astype(jnp.bfloat16)
    xl = hl_ref[0, :, 64:128].astype(jnp.bfloat16)
    xr = hr_ref[0, :, 0:64].astype(jnp.bfloat16)

    lm = jnp.dot(w1, xm, preferred_element_type=jnp.float32) + b1
    ll = jnp.dot(w1, xl, preferred_element_type=jnp.float32) + b1
    lr = jnp.dot(w1, xr, preferred_element_type=jnp.float32) + b1
    ll = jnp.where(l == 0, 0.0, ll)
    lr = jnp.where(l == nl - 1, 0.0, lr)
    lat4 = jnp.concatenate([
        jnp.transpose(ll, (1, 0)),
        jnp.transpose(lm, (1, 0)),
        jnp.transpose(lr, (1, 0)),
    ], axis=0)

    lat5 = lat5_ref[0].reshape(32, 32, 256)
    z = jnp.zeros((1, 32, 256), jnp.bfloat16)
    s18 = jnp.where(l == 0,
                    jnp.concatenate([z, lat5[0:17]], axis=0),
                    jnp.concatenate([lat5[15:32], z], axis=0))
    up = _updup(s18, 34)

    sum4 = lat4 + up.astype(jnp.float32)
    ri = jax.lax.broadcasted_iota(jnp.int32, (2176, 1), 0) // 64
    edge = ((ri == 0) & (l == 0)) | ((ri == 33) & (l == nl - 1))
    sum4 = jnp.where(edge, 0.0, sum4).astype(jnp.bfloat16)
    sum4_ref[0] = sum4.reshape(34, 64, 256)[1:33]

    rows_ref[...] = jnp.zeros_like(rows_ref)
    rows_ref[:, 1:65, :] = sum4.reshape(34, 64, 256)
    out4 = _im2col_dot(rows_ref, im_ref, w3_ref, b3_ref, 32, 64, 256)
    _to_nchw(o4_ref, out4, 32, 64)


def _call_b(f4, lat5, w1, b1, w3c, b3s):
    return pl.pallas_call(
        _p4_kernel,
        out_shape=[
            jax.ShapeDtypeStruct((2, 64, 64, 256), jnp.bfloat16),
            jax.ShapeDtypeStruct((2, 256, 64, 64), jnp.float32),
        ],
        grid=(2, 2),
        in_specs=[
            pl.BlockSpec((1, 1024, 2048), lambda n, l: (n, 0, l)),
            pl.BlockSpec((1, 1024, 128),
                         lambda n, l: (n, 0, jnp.maximum(l * 16 - 1, 0))),
            pl.BlockSpec((1, 1024, 128),
                         lambda n, l: (n, 0, jnp.minimum((l + 1) * 16, 31))),
            pl.BlockSpec((1, 1024, 256), lambda n, l: (n, 0, 0)),
            pl.BlockSpec((256, 1024), lambda n, l: (0, 0)),
            pl.BlockSpec((256, 1), lambda n, l: (0, 0)),
            pl.BlockSpec((1, 2304, 256), lambda n, l: (1, 0, 0)),
            pl.BlockSpec((1, 1, 256), lambda n, l: (1, 0, 0)),
        ],
        out_specs=[
            pl.BlockSpec((1, 32, 64, 256), lambda n, l: (n, l, 0, 0)),
            pl.BlockSpec((1, 256, 32, 64), lambda n, l: (n, 0, l, 0)),
        ],
        scratch_shapes=[
            pltpu.VMEM((34, 66, 256), jnp.bfloat16),
            pltpu.VMEM((2048, 2304), jnp.bfloat16),
        ],
        compiler_params=_params(("parallel", "parallel")),
        cost_estimate=pl.CostEstimate(
            flops=2 * 2 * (4096 * 1024 * 256 + 4096 * 2304 * 256),
            transcendentals=0,
            bytes_accessed=2 * (1024 * 4096 * 4 + 2 * 4096 * 256 * 4)),
    )(f4, f4, f4, lat5, w1, b1, w3c, b3s)


def _p3_kernel(xm_ref, hl_ref, hr_ref, s4m_ref, s4t_ref, s4b_ref,
               w1_ref, b1_ref, w3_ref, b3_ref, o3_ref, rows_ref, im_ref):
    l = pl.program_id(1)
    nl = pl.num_programs(1)
    w1 = w1_ref[...].astype(jnp.bfloat16)
    b1 = b1_ref[...]

    xm = xm_ref[0].astype(jnp.bfloat16)
    xl = hl_ref[0].astype(jnp.bfloat16)
    xr = hr_ref[0].astype(jnp.bfloat16)
    lm = jnp.dot(w1, xm, preferred_element_type=jnp.float32) + b1
    ll = jnp.dot(w1, xl, preferred_element_type=jnp.float32) + b1
    lr = jnp.dot(w1, xr, preferred_element_type=jnp.float32) + b1
    ll = jnp.where(l == 0, 0.0, ll)
    lr = jnp.where(l == nl - 1, 0.0, lr)
    lat3 = jnp.concatenate([
        jnp.transpose(ll, (1, 0)),
        jnp.transpose(lm, (1, 0)),
        jnp.transpose(lr, (1, 0)),
    ], axis=0)

    zt = jnp.where(l == 0, 0.0, s4t_ref[0]).astype(jnp.bfloat16)
    zb = jnp.where(l == nl - 1, 0.0, s4b_ref[0]).astype(jnp.bfloat16)
    s10 = jnp.concatenate([zt, s4m_ref[0], zb], axis=0)
    up = _updup(s10, 18)

    sum3 = lat3 + up.astype(jnp.float32)
    ri = jax.lax.broadcasted_iota(jnp.int32, (2304, 1), 0) // 128
    edge = ((ri == 0) & (l == 0)) | ((ri == 17) & (l == nl - 1))
    sum3 = jnp.where(edge, 0.0, sum3).astype(jnp.bfloat16)

    rows_ref[...] = jnp.zeros_like(rows_ref)
    rows_ref[:, 1:129, :] = sum3.reshape(18, 128, 256)
    out3 = _im2col_dot(rows_ref, im_ref, w3_ref, b3_ref, 16, 128, 256)
    _to_nchw(o3_ref, out3, 16, 128)


def _call_c(f3, sum4, w1, b1, w3c, b3s):
    return pl.pallas_call(
        _p3_kernel,
        out_shape=jax.ShapeDtypeStruct((2, 256, 128, 128), jnp.float32),
        grid=(2, 8),
        in_specs=[
            pl.BlockSpec((1, 512, 2048), lambda n, l: (n, 0, l)),
            pl.BlockSpec((1, 512, 128),
                         lambda n, l: (n, 0, jnp.maximum(l * 16 - 1, 0))),
            pl.BlockSpec((1, 512, 128),
                         lambda n, l: (n, 0, jnp.minimum((l + 1) * 16, 127))),
            pl.BlockSpec((1, 8, 64, 256), lambda n, l: (n, l, 0, 0)),
            pl.BlockSpec((1, 1, 64, 256),
                         lambda n, l: (n, jnp.maximum(l * 8 - 1, 0), 0, 0)),
            pl.BlockSpec((1, 1, 64, 256),
                         lambda n, l: (n, jnp.minimum(l * 8 + 8, 63), 0, 0)),
            pl.BlockSpec((256, 512), lambda n, l: (0, 0)),
            pl.BlockSpec((256, 1), lambda n, l: (0, 0)),
            pl.BlockSpec((1, 2304, 256), lambda n, l: (0, 0, 0)),
            pl.BlockSpec((1, 1, 256), lambda n, l: (0, 0, 0)),
        ],
        out_specs=pl.BlockSpec((1, 256, 16, 128), lambda n, l: (n, 0, l, 0)),
        scratch_shapes=[
            pltpu.VMEM((18, 130, 256), jnp.bfloat16),
            pltpu.VMEM((2048, 2304), jnp.bfloat16),
        ],
        compiler_params=_params(("parallel", "parallel")),
        cost_estimate=pl.CostEstimate(
            flops=2 * 2 * (16384 * 512 * 256 + 16384 * 2304 * 256),
            transcendentals=0,
            bytes_accessed=2 * (512 * 16384 * 4 + 16384 * 256 * 4)),
    )(f3, f3, f3, sum4, sum4, sum4, w1, b1, w3c, b3s)


def kernel(feat0, feat1, feat2,
           w1_0, w1_1, w1_2,
           b1_0, b1_1, b1_2,
           w3_0, w3_1, w3_2, w3_3, w3_4,
           b3_0, b3_1, b3_2, b3_3, b3_4):
    C = 256
    w3c = jnp.concatenate(
        [w.reshape(1, C, C, 9) for w in (w3_0, w3_1, w3_2, w3_3, w3_4)],
        axis=0).transpose(0, 3, 2, 1).reshape(5, 9 * C, C).astype(jnp.bfloat16)
    b3s = jnp.concatenate(
        [b.reshape(1, 1, C) for b in (b3_0, b3_1, b3_2, b3_3, b3_4)],
        axis=0).astype(jnp.float32)

    w1m = [w.reshape(C, w.shape[1]) for w in (w1_0, w1_1, w1_2)]
    b1p = [b.reshape(C, 1).astype(jnp.float32) for b in (b1_0, b1_1, b1_2)]

    f5 = feat2.reshape(2, 2048, 1024)
    f4 = feat1.reshape(2, 1024, 4096)
    f3 = feat0.reshape(2, 512, 16384)

    lat5, out5, out6, out7 = _call_a(f5, w1m[2], b1p[2], w3c, b3s)
    sum4, out4 = _call_b(f4, lat5, w1m[1], b1p[1], w3c, b3s)
    out3 = _call_c(f3, sum4, w1m[0], b1p[0], w3c, b3s)

    return [out3, out4, out5, out6, out7]
